# baseline (speedup 1.0000x reference)
import functools
import jax
import jax.numpy as jnp
from jax import lax
from jax.experimental import pallas as pl
from jax.experimental.pallas import tpu as pltpu
from jax.experimental.pallas import tpu_sc as plsc

N = 400000
G = 2000
CELLS = 1024
NSEG = CELLS * G
HSEG = NSEG // 2
NFREQ = 10
E = 10
KK = 4 * NFREQ
WPR = 224
C = 64
NCHUNK = N // C
NC, NS = 2, 16
SLAB = HSEG // NS
ZCH = 2000

_FREQS = [float(1000.0 ** (-2.0 * (i + 1) / NFREQ)) for i in range(NFREQ)]
_SIN_T = [
    [1.0, -1 / 6, 1 / 120, -1 / 5040, 1 / 362880, -1 / 39916800,
     1 / 6227020800],
    [1.0, -1 / 6, 1 / 120, -1 / 5040],
    [1.0, -1 / 6, 1 / 120],
    None,
]
_COS_T = [
    [1.0, -1 / 2, 1 / 24, -1 / 720, 1 / 40320, -1 / 3628800,
     1 / 479001600, -1 / 87178291200],
    [1.0, -1 / 2, 1 / 24, -1 / 720],
    [1.0, -1 / 2, 1 / 24],
    [1.0, -1 / 2],
]


def _tier(fi):
    return 0 if fi == 0 else 1 if fi == 1 else 2 if fi <= 4 else 3


def _poly(x2, coefs):
    r = jnp.full((16,), coefs[-1], jnp.float32)
    for c in coefs[-2::-1]:
        r = r * x2 + jnp.float32(c)
    return r


def _sincos(u, fi):
    x2 = u * u
    t = _tier(fi)
    sv = u if _SIN_T[t] is None else u * _poly(x2, _SIN_T[t])
    return sv, _poly(x2, _COS_T[t])


_sc_mesh = plsc.VectorSubcoreMesh(core_axis_name="c", subcore_axis_name="s")


@functools.partial(
    pl.kernel,
    out_type=jax.ShapeDtypeStruct((NSEG,), jnp.float32),
    mesh=_sc_mesh,
    compiler_params=pltpu.CompilerParams(needs_layout_passes=False,
                                         use_tc_tiling_on_sc=False),
    scratch_types=[
        pltpu.VMEM((G * E,), jnp.float32),
        pltpu.VMEM((C, WPR), jnp.int32),
        pltpu.VMEM((C, WPR), jnp.int32),
        pltpu.VMEM((4 * C,), jnp.int32),
        pltpu.VMEM((4 * C,), jnp.int32),
        pltpu.VMEM((C,), jnp.int32),
        pltpu.VMEM((C,), jnp.int32),
        pltpu.VMEM((C,), jnp.float32),
        pltpu.VMEM((C,), jnp.float32),
        pltpu.VMEM((16,), jnp.int32),
        pltpu.VMEM((ZCH,), jnp.float32),
        pltpu.VMEM_SHARED((HSEG,), jnp.float32),
        pltpu.SemaphoreType.DMA,
        pltpu.SemaphoreType.DMA,
        pltpu.SemaphoreType.DMA,
        pltpu.SemaphoreType.DMA,
        pltpu.SemaphoreType.DMA,
        pltpu.SemaphoreType.DMA,
    ],
)
def _sc_kernel(packed_h, lcx_h, wtab_h, expw_h, out_h,
               expwv, wr0, wr1, ib0, ib1, xb0, xb1, vb0, vb1, pb, zb, acc,
               ss0, ss1, sg0, sg1, sc0, sc1):
    cid = lax.axis_index("c")
    sid = lax.axis_index("s")
    wrows = (wr0, wr1)
    ibufs = (ib0, ib1)
    xbufs = (xb0, xb1)
    vbufs = (vb0, vb1)
    sss = (ss0, ss1)
    sgs = (sg0, sg1)
    scs = (sc0, sc1)

    pltpu.sync_copy(expw_h, expwv)

    def _zfill(j, carry):
        zb[pl.ds(j * 16, 16)] = jnp.zeros((16,), jnp.float32)
        return carry

    lax.fori_loop(0, ZCH // 16, _zfill, 0)

    def _zslab(j, carry):
        pltpu.sync_copy(zb, acc.at[pl.ds(sid * SLAB + j * ZCH, ZCH)])
        return carry

    lax.fori_loop(0, SLAB // ZCH, _zslab, 0)

    def _bs_cond(c):
        return c[0] < c[1]

    def _bs_body(c):
        lo, hi = c
        mid = (lo + hi) // 2
        pltpu.sync_copy(lcx_h.at[pl.ds(mid * C, 16)], pb)
        first = pb[...][0]
        lt = first < HSEG
        return (jnp.where(lt, mid + 1, lo), jnp.where(lt, hi, mid))

    b, _ = lax.while_loop(_bs_cond, _bs_body, (jnp.int32(0), jnp.int32(NCHUNK)))
    start = jnp.where(cid == 0, 0, jnp.maximum(b - 1, 0))
    count = jnp.where(cid == 0, b, NCHUNK - jnp.maximum(b - 1, 0))
    n = (count - sid + NS - 1) // NS
    seg0 = cid * HSEG
    other = cid != 0

    plsc.subcore_barrier()

    iota = lax.iota(jnp.int32, 16)

    def _cidx(i):
        return start + sid + i * NS

    def _small(i, p):
        return pltpu.make_async_copy(packed_h.at[_cidx(i)], ibufs[p], sss[p])

    def _rows(i, p):
        return pltpu.make_async_copy(
            wtab_h.at[ibufs[p].at[pl.ds(0, C)]], wrows[p], sgs[p])

    def _scat(p):
        return pltpu.make_async_copy(vbufs[p], acc.at[xbufs[p]], scs[p])

    def _compute(p):
        ib, wr, xb, vb = ibufs[p], wrows[p], xbufs[p], vbufs[p]

        def _sub(v, inner):
            o = v * 16
            rowv = iota + o
            gm = ib[pl.ds(o, 16)]
            seg = ib[pl.ds(C + o, 16)]
            c0v = plsc.bitcast(ib[pl.ds(2 * C + o, 16)], jnp.float32)
            c1v = plsc.bitcast(ib[pl.ds(3 * C + o, 16)], jnp.float32)
            g10 = (seg % G) * 10
            facc = [None] * E
            for eb in range(E // 2):
                w = plsc.load_gather(
                    wr, [rowv, jnp.full((16,), KK * 5 + eb, jnp.int32)])
                ba, bb = plsc.unpack(plsc.bitcast(w, jnp.bfloat16),
                                     format=plsc.PackFormat.INTERLEAVED)
                facc[2 * eb] = ba
                facc[2 * eb + 1] = bb
            pairs = [(ci, fi) for ci in (0, 1) for fi in range(NFREQ)]
            for gidx in range(5):
                acc_p = [None] * (E // 2)
                for (ci, fi) in pairs[gidx * 4:(gidx + 1) * 4]:
                    cv = c0v if ci == 0 else c1v
                    sv, cw = _sincos(cv * jnp.float32(_FREQS[fi]), fi)
                    for ptyp, s_k in ((0, sv), (1, cw)):
                        kk = ci * 20 + 2 * fi + ptyp
                        sb = plsc.pack(s_k, s_k,
                                       format=plsc.PackFormat.INTERLEAVED)
                        for ep in range(E // 2):
                            w = plsc.load_gather(
                                wr,
                                [rowv,
                                 jnp.full((16,), kk * 5 + ep, jnp.int32)])
                            t = sb * plsc.bitcast(w, jnp.bfloat16)
                            acc_p[ep] = t if acc_p[ep] is None \
                                else acc_p[ep] + t
                for ep in range(E // 2):
                    ga, gb = plsc.unpack(acc_p[ep],
                                         format=plsc.PackFormat.INTERLEAVED)
                    facc[2 * ep] = facc[2 * ep] + ga
                    facc[2 * ep + 1] = facc[2 * ep + 1] + gb
            pred = jnp.zeros((16,), jnp.float32)
            for e in range(E):
                emb = 1.0 / (1.0 + jnp.exp(-facc[e]))
                pred = pred + emb * plsc.load_gather(expwv, [g10 + e])
            keep = jnp.logical_xor(seg < HSEG, other)
            vb[pl.ds(o, 16)] = jnp.where(keep, pred, 0.0)
            xb[pl.ds(o, 16)] = jnp.clip(seg, seg0, seg0 + HSEG - 1) - seg0
            return inner

        lax.fori_loop(0, C // 16, _sub, 0)
        _scat(p).start(add=True)

    @pl.when(n > 0)
    def _prologue():
        _small(0, 0).start()
        _small(0, 0).wait()
        _rows(0, 0).start()

        @pl.when(n > 1)
        def _():
            _small(1, 1).start()

    def _pair(j, carry):
        for ph in range(2):
            i = 2 * j + ph

            @pl.when(i < n)
            def _phase():
                @pl.when(i + 1 < n)
                def _():
                    _small(i + 1, 1 - ph).wait()
                    _rows(i + 1, 1 - ph).start()

                _rows(i, ph).wait()

                @pl.when(i >= 2)
                def _():
                    _scat(ph).wait()

                _compute(ph)

                @pl.when(i + 2 < n)
                def _():
                    _small(i + 2, ph).start()

        return carry

    lax.fori_loop(0, (n + 1) // 2, _pair, 0)
    for p in (0, 1):
        @pl.when((n + 1 - p) // 2 > jnp.maximum(n - 1 - p, 0) // 2)
        def _(p=p):
            _scat(p).wait()
    plsc.subcore_barrier()
    pltpu.sync_copy(acc.at[pl.ds(sid * SLAB, SLAB)],
                    out_h.at[pl.ds(cid * HSEG + sid * SLAB, SLAB)])


def _combine_body(p_ref, b_ref, o_ref):
    o_ref[...] = p_ref[...] + b_ref[...]


def _combine(pooled, bias2d):
    return pl.pallas_call(
        _combine_body,
        grid=(CELLS // 128,),
        in_specs=[
            pl.BlockSpec((128, G), lambda i: (i, 0)),
            pl.BlockSpec((1, G), lambda i: (0, 0)),
        ],
        out_specs=pl.BlockSpec((128, G), lambda i: (i, 0)),
        out_shape=jax.ShapeDtypeStruct((CELLS, G), jnp.float32),
    )(pooled, bias2d)


def kernel(coordinates, genemapping, local_cellxgene_ix, n_cells, n_genes_mb,
           genes_oi, frag_weight1, frag_bias1, exp_weight1, exp_bias1):
    wflat = jnp.concatenate(
        [frag_weight1.reshape(G, KK * E), frag_bias1.reshape(G, E)], axis=1
    ).astype(jnp.bfloat16)
    wpad = jnp.pad(wflat, ((0, 0), (0, 2 * WPR - (KK + 1) * E)))
    wtab = lax.bitcast_convert_type(wpad.reshape(G, WPR, 2), jnp.int32)
    ci = lax.bitcast_convert_type(coordinates, jnp.int32)
    packed = jnp.concatenate(
        [genemapping.reshape(NCHUNK, C), local_cellxgene_ix.reshape(NCHUNK, C),
         ci[:, 0].reshape(NCHUNK, C), ci[:, 1].reshape(NCHUNK, C)], axis=1)
    expwflat = jnp.take(exp_weight1, genes_oi, axis=0).reshape(-1)
    pooled = _sc_kernel(packed, local_cellxgene_ix, wtab, expwflat)
    bias2d = jnp.take(exp_bias1, genes_oi, axis=0).reshape(1, G)
    return _combine(pooled.reshape(CELLS, G), bias2d)


# speedup vs baseline: 1.5107x; 1.5107x over previous
import functools
import jax
import jax.numpy as jnp
from jax import lax
from jax.experimental import pallas as pl
from jax.experimental.pallas import tpu as pltpu
from jax.experimental.pallas import tpu_sc as plsc

N = 400000
G = 2000
CELLS = 1024
NSEG = CELLS * G
HSEG = NSEG // 2
NFREQ = 10
E = 10
KK = 4 * NFREQ
WPR = 224
C = 64
NCHUNK = N // C
NC, NS = 2, 16
SLAB = HSEG // NS
ZCH = 2000

_FREQS = [float(1000.0 ** (-2.0 * (i + 1) / NFREQ)) for i in range(NFREQ)]
_SIN_T = [
    [1.0, -1 / 6, 1 / 120, -1 / 5040, 1 / 362880, -1 / 39916800,
     1 / 6227020800],
    [1.0, -1 / 6, 1 / 120, -1 / 5040],
    [1.0, -1 / 6, 1 / 120],
    None,
]
_COS_T = [
    [1.0, -1 / 2, 1 / 24, -1 / 720, 1 / 40320, -1 / 3628800,
     1 / 479001600, -1 / 87178291200],
    [1.0, -1 / 2, 1 / 24, -1 / 720],
    [1.0, -1 / 2, 1 / 24],
    [1.0, -1 / 2],
]


def _tier(fi):
    return 0 if fi == 0 else 1 if fi == 1 else 2 if fi <= 4 else 3


def _poly(x2, coefs):
    r = jnp.full((16,), coefs[-1], jnp.float32)
    for c in coefs[-2::-1]:
        r = r * x2 + jnp.float32(c)
    return r


def _sincos(u, fi):
    x2 = u * u
    t = _tier(fi)
    sv = u if _SIN_T[t] is None else u * _poly(x2, _SIN_T[t])
    return sv, _poly(x2, _COS_T[t])


_sc_mesh = plsc.VectorSubcoreMesh(core_axis_name="c", subcore_axis_name="s")


@functools.partial(
    pl.kernel,
    out_type=jax.ShapeDtypeStruct((NSEG,), jnp.float32),
    mesh=_sc_mesh,
    compiler_params=pltpu.CompilerParams(needs_layout_passes=False,
                                         use_tc_tiling_on_sc=False),
    scratch_types=[
        pltpu.VMEM((G * E,), jnp.float32),
        pltpu.VMEM((C, WPR), jnp.int32),
        pltpu.VMEM((C, WPR), jnp.int32),
        pltpu.VMEM((4 * C,), jnp.int32),
        pltpu.VMEM((4 * C,), jnp.int32),
        pltpu.VMEM((C,), jnp.int32),
        pltpu.VMEM((C,), jnp.int32),
        pltpu.VMEM((C,), jnp.float32),
        pltpu.VMEM((C,), jnp.float32),
        pltpu.VMEM((16,), jnp.int32),
        pltpu.VMEM((ZCH,), jnp.float32),
        pltpu.VMEM_SHARED((HSEG,), jnp.float32),
        pltpu.SemaphoreType.DMA,
        pltpu.SemaphoreType.DMA,
        pltpu.SemaphoreType.DMA,
        pltpu.SemaphoreType.DMA,
        pltpu.SemaphoreType.DMA,
        pltpu.SemaphoreType.DMA,
    ],
)
def _sc_kernel(packed_h, lcx_h, wtab_h, expw_h, out_h,
               expwv, wr0, wr1, ib0, ib1, xb0, xb1, vb0, vb1, pb, zb, acc,
               ss0, ss1, sg0, sg1, sc0, sc1):
    cid = lax.axis_index("c")
    sid = lax.axis_index("s")
    wrows = (wr0, wr1)
    ibufs = (ib0, ib1)
    xbufs = (xb0, xb1)
    vbufs = (vb0, vb1)
    sss = (ss0, ss1)
    sgs = (sg0, sg1)
    scs = (sc0, sc1)

    pltpu.sync_copy(expw_h, expwv)

    def _zfill(j, carry):
        zb[pl.ds(j * 16, 16)] = jnp.zeros((16,), jnp.float32)
        return carry

    lax.fori_loop(0, ZCH // 16, _zfill, 0)

    def _zslab(j, carry):
        pltpu.sync_copy(zb, acc.at[pl.ds(sid * SLAB + j * ZCH, ZCH)])
        return carry

    lax.fori_loop(0, SLAB // ZCH, _zslab, 0)

    def _bs_cond(c):
        return c[0] < c[1]

    def _bs_body(c):
        lo, hi = c
        mid = (lo + hi) // 2
        pltpu.sync_copy(lcx_h.at[pl.ds(mid * C, 16)], pb)
        first = pb[...][0]
        lt = first < HSEG
        return (jnp.where(lt, mid + 1, lo), jnp.where(lt, hi, mid))

    b, _ = lax.while_loop(_bs_cond, _bs_body, (jnp.int32(0), jnp.int32(NCHUNK)))
    start = jnp.where(cid == 0, 0, jnp.maximum(b - 1, 0))
    count = jnp.where(cid == 0, b, NCHUNK - jnp.maximum(b - 1, 0))
    n = (count - sid + NS - 1) // NS
    seg0 = cid * HSEG
    other = cid != 0

    plsc.subcore_barrier()

    iota = lax.iota(jnp.int32, 16)

    def _cidx(i):
        return start + sid + i * NS

    def _small(i, p):
        return pltpu.make_async_copy(packed_h.at[_cidx(i)], ibufs[p], sss[p])

    def _rows(i, p):
        return pltpu.make_async_copy(
            wtab_h.at[ibufs[p].at[pl.ds(0, C)]], wrows[p], sgs[p])

    def _scat(p):
        return pltpu.make_async_copy(vbufs[p], acc.at[xbufs[p]], scs[p])

    def _compute(p):
        ib, wr, xb, vb = ibufs[p], wrows[p], xbufs[p], vbufs[p]

        def _sub(v, inner):
            o = v * 16
            rowv = iota + o
            gm = ib[pl.ds(o, 16)]
            seg = ib[pl.ds(C + o, 16)]
            c0v = plsc.bitcast(ib[pl.ds(2 * C + o, 16)], jnp.float32)
            c1v = plsc.bitcast(ib[pl.ds(3 * C + o, 16)], jnp.float32)
            svec = gm & 15
            g10 = (seg % G) * 10
            zerov = jnp.zeros((16,), jnp.int32)
            basev = rowv * WPR + svec

            def _gw(col):
                return plsc.load_gather(wr, [zerov, basev + col])

            facc = [None] * E
            for eb in range(E // 2):
                w = _gw(KK * 5 + eb)
                ba, bb = plsc.unpack(plsc.bitcast(w, jnp.bfloat16),
                                     format=plsc.PackFormat.INTERLEAVED)
                facc[2 * eb] = ba
                facc[2 * eb + 1] = bb
            pairs = [(ci, fi) for ci in (0, 1) for fi in range(NFREQ)]
            for gidx in range(5):
                acc_p = [None] * (E // 2)
                for (ci, fi) in pairs[gidx * 4:(gidx + 1) * 4]:
                    cv = c0v if ci == 0 else c1v
                    sv, cw = _sincos(cv * jnp.float32(_FREQS[fi]), fi)
                    for ptyp, s_k in ((0, sv), (1, cw)):
                        kk = ci * 20 + 2 * fi + ptyp
                        sb = plsc.pack(s_k, s_k,
                                       format=plsc.PackFormat.INTERLEAVED)
                        for ep in range(E // 2):
                            t = sb * plsc.bitcast(_gw(kk * 5 + ep),
                                                  jnp.bfloat16)
                            acc_p[ep] = t if acc_p[ep] is None \
                                else acc_p[ep] + t
                for ep in range(E // 2):
                    ga, gb = plsc.unpack(acc_p[ep],
                                         format=plsc.PackFormat.INTERLEAVED)
                    facc[2 * ep] = facc[2 * ep] + ga
                    facc[2 * ep + 1] = facc[2 * ep + 1] + gb
            pred = jnp.zeros((16,), jnp.float32)
            for e in range(E):
                emb = 1.0 / (1.0 + jnp.exp(-facc[e]))
                pred = pred + emb * plsc.load_gather(expwv, [g10 + e])
            keep = jnp.logical_xor(seg < HSEG, other)
            vb[pl.ds(o, 16)] = jnp.where(keep, pred, 0.0)
            xb[pl.ds(o, 16)] = jnp.clip(seg, seg0, seg0 + HSEG - 1) - seg0
            return inner

        lax.fori_loop(0, C // 16, _sub, 0)
        _scat(p).start(add=True)

    @pl.when(n > 0)
    def _prologue():
        _small(0, 0).start()
        _small(0, 0).wait()
        _rows(0, 0).start()

        @pl.when(n > 1)
        def _():
            _small(1, 1).start()

    def _pair(j, carry):
        for ph in range(2):
            i = 2 * j + ph

            @pl.when(i < n)
            def _phase():
                @pl.when(i + 1 < n)
                def _():
                    _small(i + 1, 1 - ph).wait()
                    _rows(i + 1, 1 - ph).start()

                _rows(i, ph).wait()

                @pl.when(i >= 2)
                def _():
                    _scat(ph).wait()

                _compute(ph)

                @pl.when(i + 2 < n)
                def _():
                    _small(i + 2, ph).start()

        return carry

    lax.fori_loop(0, (n + 1) // 2, _pair, 0)
    for p in (0, 1):
        @pl.when((n + 1 - p) // 2 > jnp.maximum(n - 1 - p, 0) // 2)
        def _(p=p):
            _scat(p).wait()
    plsc.subcore_barrier()
    pltpu.sync_copy(acc.at[pl.ds(sid * SLAB, SLAB)],
                    out_h.at[pl.ds(cid * HSEG + sid * SLAB, SLAB)])


def _combine_body(p_ref, b_ref, o_ref):
    o_ref[...] = p_ref[...] + b_ref[...]


def _combine(pooled, bias2d):
    return pl.pallas_call(
        _combine_body,
        grid=(CELLS // 128,),
        in_specs=[
            pl.BlockSpec((128, G), lambda i: (i, 0)),
            pl.BlockSpec((1, G), lambda i: (0, 0)),
        ],
        out_specs=pl.BlockSpec((128, G), lambda i: (i, 0)),
        out_shape=jax.ShapeDtypeStruct((CELLS, G), jnp.float32),
    )(pooled, bias2d)


def kernel(coordinates, genemapping, local_cellxgene_ix, n_cells, n_genes_mb,
           genes_oi, frag_weight1, frag_bias1, exp_weight1, exp_bias1):
    wflat = jnp.concatenate(
        [frag_weight1.reshape(G, KK * E), frag_bias1.reshape(G, E)], axis=1
    ).astype(jnp.bfloat16)
    shift = (jnp.arange(G, dtype=jnp.int32) % 16) * 2
    cols = jnp.arange(2 * WPR, dtype=jnp.int32)[None, :] - shift[:, None]
    valid = (cols >= 0) & (cols < (KK + 1) * E)
    wpad = jnp.pad(wflat, ((0, 0), (0, 2 * WPR - (KK + 1) * E)))
    wsh = jnp.where(valid, jnp.take_along_axis(
        wpad, jnp.clip(cols, 0, 2 * WPR - 1), axis=1), jnp.bfloat16(0))
    wtab = lax.bitcast_convert_type(wsh.reshape(G, WPR, 2), jnp.int32)
    ci = lax.bitcast_convert_type(coordinates, jnp.int32)
    packed = jnp.concatenate(
        [genemapping.reshape(NCHUNK, C), local_cellxgene_ix.reshape(NCHUNK, C),
         ci[:, 0].reshape(NCHUNK, C), ci[:, 1].reshape(NCHUNK, C)], axis=1)
    expwflat = jnp.take(exp_weight1, genes_oi, axis=0).reshape(-1)
    pooled = _sc_kernel(packed, local_cellxgene_ix, wtab, expwflat)
    bias2d = jnp.take(exp_bias1, genes_oi, axis=0).reshape(1, G)
    return _combine(pooled.reshape(CELLS, G), bias2d)


# speedup vs baseline: 1.5371x; 1.0174x over previous
import functools
import jax
import jax.numpy as jnp
from jax import lax
from jax.experimental import pallas as pl
from jax.experimental.pallas import tpu as pltpu
from jax.experimental.pallas import tpu_sc as plsc

N = 400000
G = 2000
CELLS = 1024
NSEG = CELLS * G
HSEG = NSEG // 2
NFREQ = 10
E = 10
KK = 4 * NFREQ
WPR = 224
C = 80
NCHUNK = N // C
NC, NS = 2, 16
SLAB = HSEG // NS
ZCH = 2000

_FREQS = [float(1000.0 ** (-2.0 * (i + 1) / NFREQ)) for i in range(NFREQ)]
_SIN_T = [
    [1.0, -1 / 6, 1 / 120, -1 / 5040, 1 / 362880, -1 / 39916800,
     1 / 6227020800],
    [1.0, -1 / 6, 1 / 120, -1 / 5040],
    [1.0, -1 / 6, 1 / 120],
    None,
]
_COS_T = [
    [1.0, -1 / 2, 1 / 24, -1 / 720, 1 / 40320, -1 / 3628800,
     1 / 479001600, -1 / 87178291200],
    [1.0, -1 / 2, 1 / 24, -1 / 720],
    [1.0, -1 / 2, 1 / 24],
    [1.0, -1 / 2],
]


def _tier(fi):
    return 0 if fi == 0 else 1 if fi == 1 else 2 if fi <= 4 else 3


def _poly(x2, coefs):
    r = jnp.full((16,), coefs[-1], jnp.float32)
    for c in coefs[-2::-1]:
        r = r * x2 + jnp.float32(c)
    return r


def _sincos(u, fi):
    x2 = u * u
    t = _tier(fi)
    sv = u if _SIN_T[t] is None else u * _poly(x2, _SIN_T[t])
    return sv, _poly(x2, _COS_T[t])


_sc_mesh = plsc.VectorSubcoreMesh(core_axis_name="c", subcore_axis_name="s")


@functools.partial(
    pl.kernel,
    out_type=jax.ShapeDtypeStruct((NSEG,), jnp.float32),
    mesh=_sc_mesh,
    compiler_params=pltpu.CompilerParams(needs_layout_passes=False,
                                         use_tc_tiling_on_sc=False),
    scratch_types=[
        pltpu.VMEM((G * E,), jnp.float32),
        pltpu.VMEM((C, WPR), jnp.int32),
        pltpu.VMEM((C, WPR), jnp.int32),
        pltpu.VMEM((4 * C,), jnp.int32),
        pltpu.VMEM((4 * C,), jnp.int32),
        pltpu.VMEM((C,), jnp.int32),
        pltpu.VMEM((C,), jnp.int32),
        pltpu.VMEM((C,), jnp.float32),
        pltpu.VMEM((C,), jnp.float32),
        pltpu.VMEM((16,), jnp.int32),
        pltpu.VMEM((ZCH,), jnp.float32),
        pltpu.VMEM_SHARED((HSEG,), jnp.float32),
        pltpu.SemaphoreType.DMA,
        pltpu.SemaphoreType.DMA,
        pltpu.SemaphoreType.DMA,
        pltpu.SemaphoreType.DMA,
        pltpu.SemaphoreType.DMA,
        pltpu.SemaphoreType.DMA,
    ],
)
def _sc_kernel(packed_h, lcx_h, wtab_h, expw_h, out_h,
               expwv, wr0, wr1, ib0, ib1, xb0, xb1, vb0, vb1, pb, zb, acc,
               ss0, ss1, sg0, sg1, sc0, sc1):
    cid = lax.axis_index("c")
    sid = lax.axis_index("s")
    wrows = (wr0, wr1)
    ibufs = (ib0, ib1)
    xbufs = (xb0, xb1)
    vbufs = (vb0, vb1)
    sss = (ss0, ss1)
    sgs = (sg0, sg1)
    scs = (sc0, sc1)

    pltpu.sync_copy(expw_h, expwv)

    def _zfill(j, carry):
        zb[pl.ds(j * 16, 16)] = jnp.zeros((16,), jnp.float32)
        return carry

    lax.fori_loop(0, ZCH // 16, _zfill, 0)

    def _zslab(j, carry):
        pltpu.sync_copy(zb, acc.at[pl.ds(sid * SLAB + j * ZCH, ZCH)])
        return carry

    lax.fori_loop(0, SLAB // ZCH, _zslab, 0)

    def _bs_cond(c):
        return c[0] < c[1]

    def _bs_body(c):
        lo, hi = c
        mid = (lo + hi) // 2
        pltpu.sync_copy(lcx_h.at[pl.ds(mid * C, 16)], pb)
        first = pb[...][0]
        lt = first < HSEG
        return (jnp.where(lt, mid + 1, lo), jnp.where(lt, hi, mid))

    b, _ = lax.while_loop(_bs_cond, _bs_body, (jnp.int32(0), jnp.int32(NCHUNK)))
    start = jnp.where(cid == 0, 0, jnp.maximum(b - 1, 0))
    count = jnp.where(cid == 0, b, NCHUNK - jnp.maximum(b - 1, 0))
    n = (count - sid + NS - 1) // NS
    seg0 = cid * HSEG
    other = cid != 0

    plsc.subcore_barrier()

    iota = lax.iota(jnp.int32, 16)

    def _cidx(i):
        return start + sid + i * NS

    def _small(i, p):
        return pltpu.make_async_copy(packed_h.at[_cidx(i)], ibufs[p], sss[p])

    def _rows(i, p):
        return pltpu.make_async_copy(
            wtab_h.at[ibufs[p].at[pl.ds(0, C)]], wrows[p], sgs[p])

    def _scat(p):
        return pltpu.make_async_copy(vbufs[p], acc.at[xbufs[p]], scs[p])

    def _compute(p):
        ib, wr, xb, vb = ibufs[p], wrows[p], xbufs[p], vbufs[p]

        def _sub(v, inner):
            o = v * 16
            rowv = iota + o
            gm = ib[pl.ds(o, 16)]
            seg = ib[pl.ds(C + o, 16)]
            c0v = plsc.bitcast(ib[pl.ds(2 * C + o, 16)], jnp.float32)
            c1v = plsc.bitcast(ib[pl.ds(3 * C + o, 16)], jnp.float32)
            svec = gm & 15
            g10 = (seg % G) * 10
            zerov = jnp.zeros((16,), jnp.int32)
            basev = rowv * WPR + svec

            def _gw(col):
                return plsc.load_gather(wr, [zerov, basev + col])

            facc = [None] * E
            for eb in range(E // 2):
                w = _gw(KK * 5 + eb)
                ba, bb = plsc.unpack(plsc.bitcast(w, jnp.bfloat16),
                                     format=plsc.PackFormat.INTERLEAVED)
                facc[2 * eb] = ba
                facc[2 * eb + 1] = bb
            pairs = [(ci, fi) for ci in (0, 1) for fi in range(NFREQ)]
            for gidx in range(5):
                acc_p = [None] * (E // 2)
                for (ci, fi) in pairs[gidx * 4:(gidx + 1) * 4]:
                    cv = c0v if ci == 0 else c1v
                    sv, cw = _sincos(cv * jnp.float32(_FREQS[fi]), fi)
                    for ptyp, s_k in ((0, sv), (1, cw)):
                        kk = ci * 20 + 2 * fi + ptyp
                        sb = plsc.pack(s_k, s_k,
                                       format=plsc.PackFormat.INTERLEAVED)
                        for ep in range(E // 2):
                            t = sb * plsc.bitcast(_gw(kk * 5 + ep),
                                                  jnp.bfloat16)
                            acc_p[ep] = t if acc_p[ep] is None \
                                else acc_p[ep] + t
                for ep in range(E // 2):
                    ga, gb = plsc.unpack(acc_p[ep],
                                         format=plsc.PackFormat.INTERLEAVED)
                    facc[2 * ep] = facc[2 * ep] + ga
                    facc[2 * ep + 1] = facc[2 * ep + 1] + gb
            pred = jnp.zeros((16,), jnp.float32)
            for e in range(E):
                emb = 1.0 / (1.0 + jnp.exp(-facc[e]))
                pred = pred + emb * plsc.load_gather(expwv, [g10 + e])
            keep = jnp.logical_xor(seg < HSEG, other)
            vb[pl.ds(o, 16)] = jnp.where(keep, pred, 0.0)
            xb[pl.ds(o, 16)] = jnp.clip(seg, seg0, seg0 + HSEG - 1) - seg0
            return inner

        lax.fori_loop(0, C // 16, _sub, 0)
        _scat(p).start(add=True)

    @pl.when(n > 0)
    def _prologue():
        _small(0, 0).start()
        _small(0, 0).wait()
        _rows(0, 0).start()

        @pl.when(n > 1)
        def _():
            _small(1, 1).start()

    def _pair(j, carry):
        for ph in range(2):
            i = 2 * j + ph

            @pl.when(i < n)
            def _phase():
                @pl.when(i + 1 < n)
                def _():
                    _small(i + 1, 1 - ph).wait()
                    _rows(i + 1, 1 - ph).start()

                _rows(i, ph).wait()

                @pl.when(i >= 2)
                def _():
                    _scat(ph).wait()

                _compute(ph)

                @pl.when(i + 2 < n)
                def _():
                    _small(i + 2, ph).start()

        return carry

    lax.fori_loop(0, (n + 1) // 2, _pair, 0)
    for p in (0, 1):
        @pl.when((n + 1 - p) // 2 > jnp.maximum(n - 1 - p, 0) // 2)
        def _(p=p):
            _scat(p).wait()
    plsc.subcore_barrier()
    pltpu.sync_copy(acc.at[pl.ds(sid * SLAB, SLAB)],
                    out_h.at[pl.ds(cid * HSEG + sid * SLAB, SLAB)])


def _combine_body(p_ref, b_ref, o_ref):
    o_ref[...] = p_ref[...] + b_ref[...]


def _combine(pooled, bias2d):
    return pl.pallas_call(
        _combine_body,
        grid=(CELLS // 128,),
        in_specs=[
            pl.BlockSpec((128, G), lambda i: (i, 0)),
            pl.BlockSpec((1, G), lambda i: (0, 0)),
        ],
        out_specs=pl.BlockSpec((128, G), lambda i: (i, 0)),
        out_shape=jax.ShapeDtypeStruct((CELLS, G), jnp.float32),
    )(pooled, bias2d)


def kernel(coordinates, genemapping, local_cellxgene_ix, n_cells, n_genes_mb,
           genes_oi, frag_weight1, frag_bias1, exp_weight1, exp_bias1):
    wflat = jnp.concatenate(
        [frag_weight1.reshape(G, KK * E), frag_bias1.reshape(G, E)], axis=1
    ).astype(jnp.bfloat16)
    shift = (jnp.arange(G, dtype=jnp.int32) % 16) * 2
    cols = jnp.arange(2 * WPR, dtype=jnp.int32)[None, :] - shift[:, None]
    valid = (cols >= 0) & (cols < (KK + 1) * E)
    wpad = jnp.pad(wflat, ((0, 0), (0, 2 * WPR - (KK + 1) * E)))
    wsh = jnp.where(valid, jnp.take_along_axis(
        wpad, jnp.clip(cols, 0, 2 * WPR - 1), axis=1), jnp.bfloat16(0))
    wtab = lax.bitcast_convert_type(wsh.reshape(G, WPR, 2), jnp.int32)
    ci = lax.bitcast_convert_type(coordinates, jnp.int32)
    packed = jnp.concatenate(
        [genemapping.reshape(NCHUNK, C), local_cellxgene_ix.reshape(NCHUNK, C),
         ci[:, 0].reshape(NCHUNK, C), ci[:, 1].reshape(NCHUNK, C)], axis=1)
    expwflat = jnp.take(exp_weight1, genes_oi, axis=0).reshape(-1)
    pooled = _sc_kernel(packed, local_cellxgene_ix, wtab, expwflat)
    bias2d = jnp.take(exp_bias1, genes_oi, axis=0).reshape(1, G)
    return _combine(pooled.reshape(CELLS, G), bias2d)


# speedup vs baseline: 2.7033x; 1.7587x over previous
import functools
import jax
import jax.numpy as jnp
from jax import lax
from jax.experimental import pallas as pl
from jax.experimental.pallas import tpu as pltpu
from jax.experimental.pallas import tpu_sc as plsc

N = 400000
G = 2000
CELLS = 1024
NSEG = CELLS * G
HSEG = NSEG // 2
NFREQ = 10
E = 10
KK = 4 * NFREQ
WPR = 144
C = 80
NCHUNK = N // C
NC, NS = 2, 16
SLAB = HSEG // NS
ZCH = 2000

_FREQS = [float(1000.0 ** (-2.0 * (i + 1) / NFREQ)) for i in range(NFREQ)]
_SIN_T = [
    [1.0, -1 / 6, 1 / 120, -1 / 5040, 1 / 362880, -1 / 39916800,
     1 / 6227020800],
    [1.0, -1 / 6, 1 / 120, -1 / 5040],
    [1.0, -1 / 6, 1 / 120],
    None,
]
_COS_T = [
    [1.0, -1 / 2, 1 / 24, -1 / 720, 1 / 40320, -1 / 3628800,
     1 / 479001600, -1 / 87178291200],
    [1.0, -1 / 2, 1 / 24, -1 / 720],
    [1.0, -1 / 2, 1 / 24],
    [1.0, -1 / 2],
]


def _tier(fi):
    return 0 if fi == 0 else 1 if fi == 1 else 2 if fi <= 4 else 3


_SLOTS = [(0, 2), (1, 3), (4, 6), (5, 7), (8, None), (9, None)]


def _poly(x2, coefs):
    r = jnp.full((16,), coefs[-1], jnp.float32)
    for c in coefs[-2::-1]:
        r = r * x2 + jnp.float32(c)
    return r


def _sincos(u, fi):
    x2 = u * u
    t = _tier(fi)
    sv = u if _SIN_T[t] is None else u * _poly(x2, _SIN_T[t])
    return sv, _poly(x2, _COS_T[t])


_sc_mesh = plsc.VectorSubcoreMesh(core_axis_name="c", subcore_axis_name="s")


@functools.partial(
    pl.kernel,
    out_type=jax.ShapeDtypeStruct((NSEG,), jnp.float32),
    mesh=_sc_mesh,
    compiler_params=pltpu.CompilerParams(needs_layout_passes=False,
                                         use_tc_tiling_on_sc=False),
    scratch_types=[
        pltpu.VMEM((G * E,), jnp.float32),
        pltpu.VMEM((C, WPR), jnp.int32),
        pltpu.VMEM((C, WPR), jnp.int32),
        pltpu.VMEM((4 * C,), jnp.int32),
        pltpu.VMEM((4 * C,), jnp.int32),
        pltpu.VMEM((C,), jnp.int32),
        pltpu.VMEM((C,), jnp.int32),
        pltpu.VMEM((C,), jnp.float32),
        pltpu.VMEM((C,), jnp.float32),
        pltpu.VMEM((16,), jnp.int32),
        pltpu.VMEM((ZCH,), jnp.float32),
        pltpu.VMEM_SHARED((HSEG,), jnp.float32),
        pltpu.SemaphoreType.DMA,
        pltpu.SemaphoreType.DMA,
        pltpu.SemaphoreType.DMA,
        pltpu.SemaphoreType.DMA,
        pltpu.SemaphoreType.DMA,
        pltpu.SemaphoreType.DMA,
    ],
)
def _sc_kernel(packed_h, lcx_h, wtab_h, expw_h, out_h,
               expwv, wr0, wr1, ib0, ib1, xb0, xb1, vb0, vb1, pb, zb, acc,
               ss0, ss1, sg0, sg1, sc0, sc1):
    cid = lax.axis_index("c")
    sid = lax.axis_index("s")
    wrows = (wr0, wr1)
    ibufs = (ib0, ib1)
    xbufs = (xb0, xb1)
    vbufs = (vb0, vb1)
    sss = (ss0, ss1)
    sgs = (sg0, sg1)
    scs = (sc0, sc1)

    pltpu.sync_copy(expw_h, expwv)

    def _zfill(j, carry):
        zb[pl.ds(j * 16, 16)] = jnp.zeros((16,), jnp.float32)
        return carry

    lax.fori_loop(0, ZCH // 16, _zfill, 0)

    def _zslab(j, carry):
        pltpu.sync_copy(zb, acc.at[pl.ds(sid * SLAB + j * ZCH, ZCH)])
        return carry

    lax.fori_loop(0, SLAB // ZCH, _zslab, 0)

    def _bs_cond(c):
        return c[0] < c[1]

    def _bs_body(c):
        lo, hi = c
        mid = (lo + hi) // 2
        pltpu.sync_copy(lcx_h.at[pl.ds(mid * C, 16)], pb)
        first = pb[...][0]
        lt = first < HSEG
        return (jnp.where(lt, mid + 1, lo), jnp.where(lt, hi, mid))

    b, _ = lax.while_loop(_bs_cond, _bs_body, (jnp.int32(0), jnp.int32(NCHUNK)))
    start = jnp.where(cid == 0, 0, jnp.maximum(b - 1, 0))
    count = jnp.where(cid == 0, b, NCHUNK - jnp.maximum(b - 1, 0))
    n = (count - sid + NS - 1) // NS
    seg0 = cid * HSEG
    other = cid != 0

    plsc.subcore_barrier()

    iota = lax.iota(jnp.int32, 16)

    def _cidx(i):
        return start + sid + i * NS

    def _small(i, p):
        return pltpu.make_async_copy(packed_h.at[_cidx(i)], ibufs[p], sss[p])

    def _rows(i, p):
        return pltpu.make_async_copy(
            wtab_h.at[ibufs[p].at[pl.ds(0, C)]], wrows[p], sgs[p])

    def _scat(p):
        return pltpu.make_async_copy(vbufs[p], acc.at[xbufs[p]], scs[p])

    def _compute(p):
        ib, wr, xb, vb = ibufs[p], wrows[p], xbufs[p], vbufs[p]

        def _sub(v, inner):
            o = v * 16
            rowv = iota + o
            gm = ib[pl.ds(o, 16)]
            seg = ib[pl.ds(C + o, 16)]
            c0v = plsc.bitcast(ib[pl.ds(2 * C + o, 16)], jnp.float32)
            c1v = plsc.bitcast(ib[pl.ds(3 * C + o, 16)], jnp.float32)
            svec = gm & 15
            g10 = (seg % G) * 10
            zerov = jnp.zeros((16,), jnp.int32)
            basev = rowv * WPR + svec

            def _gw(col):
                return plsc.load_gather(wr, [zerov, basev + col])

            facc = [None] * E
            for eb in range(E // 2):
                w = _gw(40 * 3 + eb)
                ba, bb = plsc.unpack(plsc.bitcast(w, jnp.bfloat16),
                                     format=plsc.PackFormat.INTERLEAVED)
                facc[2 * eb] = ba
                facc[2 * eb + 1] = bb
            pairs = [(ci, fi) for ci in (0, 1) for fi in range(NFREQ)]
            for gidx in range(5):
                acc_p = [None] * 6
                for (ci, fi) in pairs[gidx * 4:(gidx + 1) * 4]:
                    cv = c0v if ci == 0 else c1v
                    sv, cw = _sincos(cv * jnp.float32(_FREQS[fi]), fi)
                    for ptyp, s_k in ((0, sv), (1, cw)):
                        kk = ci * 20 + 2 * fi + ptyp
                        sb = plsc.pack(s_k, s_k,
                                       format=plsc.PackFormat.INTERLEAVED)
                        for j in range(3):
                            f8v = plsc.bitcast(_gw(kk * 3 + j),
                                               jnp.float8_e4m3fn)
                            ua, ub = plsc.unpack(
                                f8v, format=plsc.PackFormat.INTERLEAVED,
                                preferred_element_type=jnp.bfloat16)
                            for s2, uu in ((2 * j, ua), (2 * j + 1, ub)):
                                t = sb * uu
                                acc_p[s2] = t if acc_p[s2] is None \
                                    else acc_p[s2] + t
                for s2, (elo, ehi) in enumerate(_SLOTS):
                    ga, gb = plsc.unpack(acc_p[s2],
                                         format=plsc.PackFormat.INTERLEAVED)
                    facc[elo] = facc[elo] + ga
                    if ehi is not None:
                        facc[ehi] = facc[ehi] + gb
            pred = jnp.zeros((16,), jnp.float32)
            for e in range(E):
                emb = 1.0 / (1.0 + jnp.exp(-facc[e]))
                pred = pred + emb * plsc.load_gather(expwv, [g10 + e])
            keep = jnp.logical_xor(seg < HSEG, other)
            vb[pl.ds(o, 16)] = jnp.where(keep, pred, 0.0)
            xb[pl.ds(o, 16)] = jnp.clip(seg, seg0, seg0 + HSEG - 1) - seg0
            return inner

        lax.fori_loop(0, C // 16, _sub, 0)
        _scat(p).start(add=True)

    @pl.when(n > 0)
    def _prologue():
        _small(0, 0).start()
        _small(0, 0).wait()
        _rows(0, 0).start()

        @pl.when(n > 1)
        def _():
            _small(1, 1).start()

    def _pair(j, carry):
        for ph in range(2):
            i = 2 * j + ph

            @pl.when(i < n)
            def _phase():
                @pl.when(i + 1 < n)
                def _():
                    _small(i + 1, 1 - ph).wait()
                    _rows(i + 1, 1 - ph).start()

                _rows(i, ph).wait()

                @pl.when(i >= 2)
                def _():
                    _scat(ph).wait()

                _compute(ph)

                @pl.when(i + 2 < n)
                def _():
                    _small(i + 2, ph).start()

        return carry

    lax.fori_loop(0, (n + 1) // 2, _pair, 0)
    for p in (0, 1):
        @pl.when((n + 1 - p) // 2 > jnp.maximum(n - 1 - p, 0) // 2)
        def _(p=p):
            _scat(p).wait()
    plsc.subcore_barrier()
    pltpu.sync_copy(acc.at[pl.ds(sid * SLAB, SLAB)],
                    out_h.at[pl.ds(cid * HSEG + sid * SLAB, SLAB)])


def _combine_body(p_ref, b_ref, o_ref):
    o_ref[...] = p_ref[...] + b_ref[...]


def _combine(pooled, bias2d):
    return pl.pallas_call(
        _combine_body,
        grid=(CELLS // 128,),
        in_specs=[
            pl.BlockSpec((128, G), lambda i: (i, 0)),
            pl.BlockSpec((1, G), lambda i: (0, 0)),
        ],
        out_specs=pl.BlockSpec((128, G), lambda i: (i, 0)),
        out_shape=jax.ShapeDtypeStruct((CELLS, G), jnp.float32),
    )(pooled, bias2d)


def kernel(coordinates, genemapping, local_cellxgene_ix, n_cells, n_genes_mb,
           genes_oi, frag_weight1, frag_bias1, exp_weight1, exp_bias1):
    wq = jnp.pad(frag_weight1.astype(jnp.float8_e4m3fn),
                 ((0, 0), (0, 0), (0, 2)))
    wq_i = lax.bitcast_convert_type(
        wq.reshape(G, KK * 3, 4), jnp.int32)
    b_i = lax.bitcast_convert_type(
        frag_bias1.astype(jnp.bfloat16).reshape(G, E // 2, 2), jnp.int32)
    row = jnp.concatenate(
        [wq_i, b_i, jnp.zeros((G, WPR - KK * 3 - E // 2), jnp.int32)], axis=1)
    shift = jnp.arange(G, dtype=jnp.int32) % 16
    cols = jnp.arange(WPR, dtype=jnp.int32)[None, :] - shift[:, None]
    valid = (cols >= 0) & (cols < KK * 3 + E // 2)
    wtab = jnp.where(valid, jnp.take_along_axis(
        row, jnp.clip(cols, 0, WPR - 1), axis=1), 0)
    ci = lax.bitcast_convert_type(coordinates, jnp.int32)
    packed = jnp.concatenate(
        [genemapping.reshape(NCHUNK, C), local_cellxgene_ix.reshape(NCHUNK, C),
         ci[:, 0].reshape(NCHUNK, C), ci[:, 1].reshape(NCHUNK, C)], axis=1)
    expwflat = jnp.take(exp_weight1, genes_oi, axis=0).reshape(-1)
    pooled = _sc_kernel(packed, local_cellxgene_ix, wtab, expwflat)
    bias2d = jnp.take(exp_bias1, genes_oi, axis=0).reshape(1, G)
    return _combine(pooled.reshape(CELLS, G), bias2d)


# speedup vs baseline: 2.9944x; 1.1077x over previous
import functools
import jax
import jax.numpy as jnp
from jax import lax
from jax.experimental import pallas as pl
from jax.experimental.pallas import tpu as pltpu
from jax.experimental.pallas import tpu_sc as plsc

N = 400000
G = 2000
CELLS = 1024
NSEG = CELLS * G
HSEG = NSEG // 2
NFREQ = 10
E = 10
KK = 4 * NFREQ
WPR = 144
C = 128
NCHUNK = N // C
NC, NS = 2, 16
SLAB = HSEG // NS
ZCH = 2000

_FREQS = [float(1000.0 ** (-2.0 * (i + 1) / NFREQ)) for i in range(NFREQ)]
_SIN_T = [
    [1.0, -1 / 6, 1 / 120, -1 / 5040, 1 / 362880, -1 / 39916800,
     1 / 6227020800],
    [1.0, -1 / 6, 1 / 120, -1 / 5040],
    [1.0, -1 / 6, 1 / 120],
    None,
]
_COS_T = [
    [1.0, -1 / 2, 1 / 24, -1 / 720, 1 / 40320, -1 / 3628800,
     1 / 479001600, -1 / 87178291200],
    [1.0, -1 / 2, 1 / 24, -1 / 720],
    [1.0, -1 / 2, 1 / 24],
    [1.0, -1 / 2],
]


def _tier(fi):
    return 0 if fi == 0 else 1 if fi == 1 else 2 if fi <= 4 else 3


_SLOTS = [(0, 2), (1, 3), (4, 6), (5, 7), (8, None), (9, None)]


def _poly(x2, coefs):
    r = jnp.full((16,), coefs[-1], jnp.float32)
    for c in coefs[-2::-1]:
        r = r * x2 + jnp.float32(c)
    return r


def _sincos(u, fi):
    x2 = u * u
    t = _tier(fi)
    sv = u if _SIN_T[t] is None else u * _poly(x2, _SIN_T[t])
    return sv, _poly(x2, _COS_T[t])


_sc_mesh = plsc.VectorSubcoreMesh(core_axis_name="c", subcore_axis_name="s")


@functools.partial(
    pl.kernel,
    out_type=jax.ShapeDtypeStruct((NSEG,), jnp.float32),
    mesh=_sc_mesh,
    compiler_params=pltpu.CompilerParams(needs_layout_passes=False,
                                         use_tc_tiling_on_sc=False),
    scratch_types=[
        pltpu.VMEM((G * E,), jnp.float32),
        pltpu.VMEM((C, WPR), jnp.int32),
        pltpu.VMEM((C, WPR), jnp.int32),
        pltpu.VMEM((4 * C,), jnp.int32),
        pltpu.VMEM((4 * C,), jnp.int32),
        pltpu.VMEM((C,), jnp.int32),
        pltpu.VMEM((C,), jnp.int32),
        pltpu.VMEM((C,), jnp.float32),
        pltpu.VMEM((C,), jnp.float32),
        pltpu.VMEM((16,), jnp.int32),
        pltpu.VMEM((ZCH,), jnp.float32),
        pltpu.VMEM_SHARED((HSEG,), jnp.float32),
        pltpu.SemaphoreType.DMA,
        pltpu.SemaphoreType.DMA,
        pltpu.SemaphoreType.DMA,
        pltpu.SemaphoreType.DMA,
        pltpu.SemaphoreType.DMA,
        pltpu.SemaphoreType.DMA,
    ],
)
def _sc_kernel(packed_h, lcx_h, wtab_h, expw_h, out_h,
               expwv, wr0, wr1, ib0, ib1, xb0, xb1, vb0, vb1, pb, zb, acc,
               ss0, ss1, sg0, sg1, sc0, sc1):
    cid = lax.axis_index("c")
    sid = lax.axis_index("s")
    wrows = (wr0, wr1)
    ibufs = (ib0, ib1)
    xbufs = (xb0, xb1)
    vbufs = (vb0, vb1)
    sss = (ss0, ss1)
    sgs = (sg0, sg1)
    scs = (sc0, sc1)

    pltpu.sync_copy(expw_h, expwv)

    def _zfill(j, carry):
        zb[pl.ds(j * 16, 16)] = jnp.zeros((16,), jnp.float32)
        return carry

    lax.fori_loop(0, ZCH // 16, _zfill, 0)

    def _zslab(j, carry):
        pltpu.sync_copy(zb, acc.at[pl.ds(sid * SLAB + j * ZCH, ZCH)])
        return carry

    lax.fori_loop(0, SLAB // ZCH, _zslab, 0)

    def _bs_cond(c):
        return c[0] < c[1]

    def _bs_body(c):
        lo, hi = c
        mid = (lo + hi) // 2
        pltpu.sync_copy(lcx_h.at[pl.ds(mid * C, 16)], pb)
        first = pb[...][0]
        lt = first < HSEG
        return (jnp.where(lt, mid + 1, lo), jnp.where(lt, hi, mid))

    b, _ = lax.while_loop(_bs_cond, _bs_body, (jnp.int32(0), jnp.int32(NCHUNK)))
    start = jnp.where(cid == 0, 0, jnp.maximum(b - 1, 0))
    count = jnp.where(cid == 0, b, NCHUNK - jnp.maximum(b - 1, 0))
    n = (count - sid + NS - 1) // NS
    seg0 = cid * HSEG
    other = cid != 0

    plsc.subcore_barrier()

    iota = lax.iota(jnp.int32, 16)

    def _cidx(i):
        return start + sid + i * NS

    def _small(i, p):
        return pltpu.make_async_copy(packed_h.at[_cidx(i)], ibufs[p], sss[p])

    def _rows(i, p):
        return pltpu.make_async_copy(
            wtab_h.at[ibufs[p].at[pl.ds(0, C)]], wrows[p], sgs[p])

    def _scat(p):
        return pltpu.make_async_copy(vbufs[p], acc.at[xbufs[p]], scs[p])

    def _compute(p):
        ib, wr, xb, vb = ibufs[p], wrows[p], xbufs[p], vbufs[p]

        def _sub(v, inner):
            o = v * 16
            rowv = iota + o
            gm = ib[pl.ds(o, 16)]
            seg = ib[pl.ds(C + o, 16)]
            c0v = plsc.bitcast(ib[pl.ds(2 * C + o, 16)], jnp.float32)
            c1v = plsc.bitcast(ib[pl.ds(3 * C + o, 16)], jnp.float32)
            svec = gm & 15
            g10 = (seg % G) * 10
            zerov = jnp.zeros((16,), jnp.int32)
            basev = rowv * WPR + svec

            def _gw(col):
                return plsc.load_gather(wr, [zerov, basev + col])

            facc = [None] * E
            for eb in range(E // 2):
                w = _gw(40 * 3 + eb)
                ba, bb = plsc.unpack(plsc.bitcast(w, jnp.bfloat16),
                                     format=plsc.PackFormat.INTERLEAVED)
                facc[2 * eb] = ba
                facc[2 * eb + 1] = bb
            pairs = [(ci, fi) for ci in (0, 1) for fi in range(NFREQ)]
            for gidx in range(5):
                acc_p = [None] * 6
                for (ci, fi) in pairs[gidx * 4:(gidx + 1) * 4]:
                    cv = c0v if ci == 0 else c1v
                    sv, cw = _sincos(cv * jnp.float32(_FREQS[fi]), fi)
                    for ptyp, s_k in ((0, sv), (1, cw)):
                        kk = ci * 20 + 2 * fi + ptyp
                        sb = plsc.pack(s_k, s_k,
                                       format=plsc.PackFormat.INTERLEAVED)
                        for j in range(3):
                            f8v = plsc.bitcast(_gw(kk * 3 + j),
                                               jnp.float8_e4m3fn)
                            ua, ub = plsc.unpack(
                                f8v, format=plsc.PackFormat.INTERLEAVED,
                                preferred_element_type=jnp.bfloat16)
                            for s2, uu in ((2 * j, ua), (2 * j + 1, ub)):
                                t = sb * uu
                                acc_p[s2] = t if acc_p[s2] is None \
                                    else acc_p[s2] + t
                for s2, (elo, ehi) in enumerate(_SLOTS):
                    ga, gb = plsc.unpack(acc_p[s2],
                                         format=plsc.PackFormat.INTERLEAVED)
                    facc[elo] = facc[elo] + ga
                    if ehi is not None:
                        facc[ehi] = facc[ehi] + gb
            pred = jnp.zeros((16,), jnp.float32)
            for e in range(E):
                emb = 1.0 / (1.0 + jnp.exp(-facc[e]))
                pred = pred + emb * plsc.load_gather(expwv, [g10 + e])
            keep = jnp.logical_xor(seg < HSEG, other)
            vb[pl.ds(o, 16)] = jnp.where(keep, pred, 0.0)
            xb[pl.ds(o, 16)] = jnp.clip(seg, seg0, seg0 + HSEG - 1) - seg0
            return inner

        lax.fori_loop(0, C // 16, _sub, 0)
        _scat(p).start(add=True)

    @pl.when(n > 0)
    def _prologue():
        _small(0, 0).start()
        _small(0, 0).wait()
        _rows(0, 0).start()

        @pl.when(n > 1)
        def _():
            _small(1, 1).start()

    def _pair(j, carry):
        for ph in range(2):
            i = 2 * j + ph

            @pl.when(i < n)
            def _phase():
                @pl.when(i + 1 < n)
                def _():
                    _small(i + 1, 1 - ph).wait()
                    _rows(i + 1, 1 - ph).start()

                _rows(i, ph).wait()

                @pl.when(i >= 2)
                def _():
                    _scat(ph).wait()

                _compute(ph)

                @pl.when(i + 2 < n)
                def _():
                    _small(i + 2, ph).start()

        return carry

    lax.fori_loop(0, (n + 1) // 2, _pair, 0)
    for p in (0, 1):
        @pl.when((n + 1 - p) // 2 > jnp.maximum(n - 1 - p, 0) // 2)
        def _(p=p):
            _scat(p).wait()
    plsc.subcore_barrier()
    pltpu.sync_copy(acc.at[pl.ds(sid * SLAB, SLAB)],
                    out_h.at[pl.ds(cid * HSEG + sid * SLAB, SLAB)])


def _combine_body(p_ref, b_ref, o_ref):
    o_ref[...] = p_ref[...] + b_ref[...]


def _combine(pooled, bias2d):
    return pl.pallas_call(
        _combine_body,
        grid=(CELLS // 128,),
        in_specs=[
            pl.BlockSpec((128, G), lambda i: (i, 0)),
            pl.BlockSpec((1, G), lambda i: (0, 0)),
        ],
        out_specs=pl.BlockSpec((128, G), lambda i: (i, 0)),
        out_shape=jax.ShapeDtypeStruct((CELLS, G), jnp.float32),
    )(pooled, bias2d)


def kernel(coordinates, genemapping, local_cellxgene_ix, n_cells, n_genes_mb,
           genes_oi, frag_weight1, frag_bias1, exp_weight1, exp_bias1):
    wq = jnp.pad(frag_weight1.astype(jnp.float8_e4m3fn),
                 ((0, 0), (0, 0), (0, 2)))
    wq_i = lax.bitcast_convert_type(
        wq.reshape(G, KK * 3, 4), jnp.int32)
    b_i = lax.bitcast_convert_type(
        frag_bias1.astype(jnp.bfloat16).reshape(G, E // 2, 2), jnp.int32)
    row = jnp.concatenate(
        [wq_i, b_i, jnp.zeros((G, WPR - KK * 3 - E // 2), jnp.int32)], axis=1)
    shift = jnp.arange(G, dtype=jnp.int32) % 16
    cols = jnp.arange(WPR, dtype=jnp.int32)[None, :] - shift[:, None]
    valid = (cols >= 0) & (cols < KK * 3 + E // 2)
    wtab = jnp.where(valid, jnp.take_along_axis(
        row, jnp.clip(cols, 0, WPR - 1), axis=1), 0)
    ci = lax.bitcast_convert_type(coordinates, jnp.int32)
    packed = jnp.concatenate(
        [genemapping.reshape(NCHUNK, C), local_cellxgene_ix.reshape(NCHUNK, C),
         ci[:, 0].reshape(NCHUNK, C), ci[:, 1].reshape(NCHUNK, C)], axis=1)
    expwflat = jnp.take(exp_weight1, genes_oi, axis=0).reshape(-1)
    pooled = _sc_kernel(packed, local_cellxgene_ix, wtab, expwflat)
    bias2d = jnp.take(exp_bias1, genes_oi, axis=0).reshape(1, G)
    return _combine(pooled.reshape(CELLS, G), bias2d)


# speedup vs baseline: 3.1663x; 1.0574x over previous
import functools
import jax
import jax.numpy as jnp
from jax import lax
from jax.experimental import pallas as pl
from jax.experimental.pallas import tpu as pltpu
from jax.experimental.pallas import tpu_sc as plsc

N = 400000
G = 2000
CELLS = 1024
NSEG = CELLS * G
HSEG = NSEG // 2
NFREQ = 10
E = 10
KK = 4 * NFREQ
WPR = 144
C = 128
NCHUNK = N // C
NC, NS = 2, 16
SLAB = HSEG // NS
ZCH = 2000

_FREQS = [float(1000.0 ** (-2.0 * (i + 1) / NFREQ)) for i in range(NFREQ)]
_SIN_T = [
    [1.0, -1 / 6, 1 / 120, -1 / 5040, 1 / 362880, -1 / 39916800,
     1 / 6227020800],
    [1.0, -1 / 6, 1 / 120, -1 / 5040],
    [1.0, -1 / 6, 1 / 120],
    None,
]
_COS_T = [
    [1.0, -1 / 2, 1 / 24, -1 / 720, 1 / 40320, -1 / 3628800,
     1 / 479001600, -1 / 87178291200],
    [1.0, -1 / 2, 1 / 24, -1 / 720],
    [1.0, -1 / 2, 1 / 24],
    [1.0, -1 / 2],
]


def _tier(fi):
    return 0 if fi == 0 else 1 if fi == 1 else 2 if fi <= 4 else 3


_SLOTS = [(0, 2), (1, 3), (4, 6), (5, 7), (8, None), (9, None)]


def _poly(x2, coefs):
    r = jnp.full((16,), coefs[-1], jnp.float32)
    for c in coefs[-2::-1]:
        r = r * x2 + jnp.float32(c)
    return r


def _sincos(u, fi):
    x2 = u * u
    t = _tier(fi)
    sv = u if _SIN_T[t] is None else u * _poly(x2, _SIN_T[t])
    return sv, _poly(x2, _COS_T[t])


_sc_mesh = plsc.VectorSubcoreMesh(core_axis_name="c", subcore_axis_name="s")


@functools.partial(
    pl.kernel,
    out_type=jax.ShapeDtypeStruct((NSEG,), jnp.float32),
    mesh=_sc_mesh,
    compiler_params=pltpu.CompilerParams(needs_layout_passes=False,
                                         use_tc_tiling_on_sc=False),
    scratch_types=[
        pltpu.VMEM((G * E,), jnp.float32),
        pltpu.VMEM((C, WPR), jnp.int32),
        pltpu.VMEM((C, WPR), jnp.int32),
        pltpu.VMEM((4 * C,), jnp.int32),
        pltpu.VMEM((4 * C,), jnp.int32),
        pltpu.VMEM((C,), jnp.int32),
        pltpu.VMEM((C,), jnp.int32),
        pltpu.VMEM((C,), jnp.float32),
        pltpu.VMEM((C,), jnp.float32),
        pltpu.VMEM((16,), jnp.int32),
        pltpu.VMEM((ZCH,), jnp.float32),
        pltpu.VMEM_SHARED((HSEG,), jnp.float32),
        pltpu.SemaphoreType.DMA,
        pltpu.SemaphoreType.DMA,
        pltpu.SemaphoreType.DMA,
        pltpu.SemaphoreType.DMA,
        pltpu.SemaphoreType.DMA,
        pltpu.SemaphoreType.DMA,
    ],
)
def _sc_kernel(packed_h, lcx_h, wtab_h, expw_h, out_h,
               expwv, wr0, wr1, ib0, ib1, xb0, xb1, vb0, vb1, pb, zb, acc,
               ss0, ss1, sg0, sg1, sc0, sc1):
    cid = lax.axis_index("c")
    sid = lax.axis_index("s")
    wrows = (wr0, wr1)
    ibufs = (ib0, ib1)
    xbufs = (xb0, xb1)
    vbufs = (vb0, vb1)
    sss = (ss0, ss1)
    sgs = (sg0, sg1)
    scs = (sc0, sc1)

    pltpu.sync_copy(expw_h, expwv)

    def _zfill(j, carry):
        zb[pl.ds(j * 16, 16)] = jnp.zeros((16,), jnp.float32)
        return carry

    lax.fori_loop(0, ZCH // 16, _zfill, 0)

    def _zslab(j, carry):
        pltpu.sync_copy(zb, acc.at[pl.ds(sid * SLAB + j * ZCH, ZCH)])
        return carry

    lax.fori_loop(0, SLAB // ZCH, _zslab, 0)

    def _bs_cond(c):
        return c[0] < c[1]

    def _bs_body(c):
        lo, hi = c
        mid = (lo + hi) // 2
        pltpu.sync_copy(lcx_h.at[pl.ds(mid * C, 16)], pb)
        first = pb[...][0]
        lt = first < HSEG
        return (jnp.where(lt, mid + 1, lo), jnp.where(lt, hi, mid))

    b, _ = lax.while_loop(_bs_cond, _bs_body, (jnp.int32(0), jnp.int32(NCHUNK)))
    start = jnp.where(cid == 0, 0, jnp.maximum(b - 1, 0))
    count = jnp.where(cid == 0, b, NCHUNK - jnp.maximum(b - 1, 0))
    n = (count - sid + NS - 1) // NS
    seg0 = cid * HSEG
    other = cid != 0

    plsc.subcore_barrier()

    iota = lax.iota(jnp.int32, 16)

    def _cidx(i):
        return start + sid + i * NS

    def _small(i, p):
        return pltpu.make_async_copy(packed_h.at[_cidx(i)], ibufs[p], sss[p])

    def _rows(i, p):
        return pltpu.make_async_copy(
            wtab_h.at[ibufs[p].at[pl.ds(0, C)]], wrows[p], sgs[p])

    def _scat(p):
        return pltpu.make_async_copy(vbufs[p], acc.at[xbufs[p]], scs[p])

    def _compute(p):
        ib, wr, xb, vb = ibufs[p], wrows[p], xbufs[p], vbufs[p]

        def _sub(v, inner):
            o = v * 16
            rowv = iota + o
            gm = ib[pl.ds(o, 16)]
            seg = ib[pl.ds(C + o, 16)]
            c0v = plsc.bitcast(ib[pl.ds(2 * C + o, 16)], jnp.float32)
            c1v = plsc.bitcast(ib[pl.ds(3 * C + o, 16)], jnp.float32)
            svec = gm & 15
            g10 = (seg % G) * 10
            zerov = jnp.zeros((16,), jnp.int32)
            basev = rowv * WPR + svec

            def _gw(col):
                return plsc.load_gather(wr, [zerov, basev + col])

            facc = [None] * E
            for eb in range(E // 2):
                w = _gw(40 * 3 + eb)
                ba, bb = plsc.unpack(plsc.bitcast(w, jnp.bfloat16),
                                     format=plsc.PackFormat.INTERLEAVED)
                facc[2 * eb] = ba
                facc[2 * eb + 1] = bb
            pairs = [(ci, fi) for ci in (0, 1) for fi in range(NFREQ)]
            for gidx in range(5):
                acc_p = [None] * 6
                for (ci, fi) in pairs[gidx * 4:(gidx + 1) * 4]:
                    cv = c0v if ci == 0 else c1v
                    sv, cw = _sincos(cv * jnp.float32(_FREQS[fi]), fi)
                    for ptyp, s_k in ((0, sv), (1, cw)):
                        kk = ci * 20 + 2 * fi + ptyp
                        sb = plsc.pack(s_k, s_k,
                                       format=plsc.PackFormat.INTERLEAVED)
                        for j in range(3):
                            f8v = plsc.bitcast(_gw(kk * 3 + j),
                                               jnp.float8_e4m3fn)
                            ua, ub = plsc.unpack(
                                f8v, format=plsc.PackFormat.INTERLEAVED,
                                preferred_element_type=jnp.bfloat16)
                            for s2, uu in ((2 * j, ua), (2 * j + 1, ub)):
                                t = sb * uu
                                acc_p[s2] = t if acc_p[s2] is None \
                                    else acc_p[s2] + t
                for s2, (elo, ehi) in enumerate(_SLOTS):
                    ga, gb = plsc.unpack(acc_p[s2],
                                         format=plsc.PackFormat.INTERLEAVED)
                    facc[elo] = facc[elo] + ga
                    if ehi is not None:
                        facc[ehi] = facc[ehi] + gb
            pred = jnp.zeros((16,), jnp.float32)
            for e in range(E):
                emb = 1.0 / (1.0 + jnp.exp(-facc[e]))
                pred = pred + emb * plsc.load_gather(expwv, [g10 + e])
            keep = jnp.logical_xor(seg < HSEG, other)
            vb[pl.ds(o, 16)] = jnp.where(keep, pred, 0.0)
            xb[pl.ds(o, 16)] = jnp.clip(seg, seg0, seg0 + HSEG - 1) - seg0
            return inner

        lax.fori_loop(0, C // 16, _sub, 0)
        _scat(p).start(add=True)

    @pl.when(n > 0)
    def _prologue():
        _small(0, 0).start()
        _small(0, 0).wait()
        _rows(0, 0).start()

        @pl.when(n > 1)
        def _():
            _small(1, 1).start()

    def _pair(j, carry):
        for ph in range(2):
            i = 2 * j + ph

            @pl.when(i < n)
            def _phase():
                @pl.when(i + 1 < n)
                def _():
                    _small(i + 1, 1 - ph).wait()
                    _rows(i + 1, 1 - ph).start()

                _rows(i, ph).wait()

                @pl.when(i >= 2)
                def _():
                    _scat(ph).wait()

                _compute(ph)

                @pl.when(i + 2 < n)
                def _():
                    _small(i + 2, ph).start()

        return carry

    lax.fori_loop(0, (n + 1) // 2, _pair, 0)
    for p in (0, 1):
        @pl.when((n + 1 - p) // 2 > jnp.maximum(n - 1 - p, 0) // 2)
        def _(p=p):
            _scat(p).wait()
    plsc.subcore_barrier()
    pltpu.sync_copy(acc.at[pl.ds(sid * SLAB, SLAB)],
                    out_h.at[pl.ds(cid * HSEG + sid * SLAB, SLAB)])


def _combine_body(p_ref, b_ref, o_ref):
    o_ref[...] = p_ref[...] + b_ref[...]


def _combine(pooled, bias2d):
    return pl.pallas_call(
        _combine_body,
        grid=(CELLS // 128,),
        in_specs=[
            pl.BlockSpec((128, G), lambda i: (i, 0)),
            pl.BlockSpec((1, G), lambda i: (0, 0)),
        ],
        out_specs=pl.BlockSpec((128, G), lambda i: (i, 0)),
        out_shape=jax.ShapeDtypeStruct((CELLS, G), jnp.float32),
    )(pooled, bias2d)


def kernel(coordinates, genemapping, local_cellxgene_ix, n_cells, n_genes_mb,
           genes_oi, frag_weight1, frag_bias1, exp_weight1, exp_bias1):
    wq = jnp.pad(frag_weight1.astype(jnp.float8_e4m3fn),
                 ((0, 0), (0, 0), (0, 2)))
    wq_i = lax.bitcast_convert_type(
        wq.reshape(G, KK * 3, 4), jnp.int32)
    b_i = lax.bitcast_convert_type(
        frag_bias1.astype(jnp.bfloat16).reshape(G, E // 2, 2), jnp.int32)
    used = KK * 3 + E // 2
    core = jnp.concatenate([wq_i, b_i], axis=1).reshape(G // 16, 16, used)
    wtab = jnp.stack(
        [jnp.pad(core[:, r], ((0, 0), (r, WPR - used - r)))
         for r in range(16)], axis=1).reshape(G, WPR)
    ci = lax.bitcast_convert_type(coordinates, jnp.int32)
    packed = jnp.concatenate(
        [genemapping.reshape(NCHUNK, C), local_cellxgene_ix.reshape(NCHUNK, C),
         ci[:, 0].reshape(NCHUNK, C), ci[:, 1].reshape(NCHUNK, C)], axis=1)
    expwflat = jnp.take(exp_weight1, genes_oi, axis=0).reshape(-1)
    pooled = _sc_kernel(packed, local_cellxgene_ix, wtab, expwflat)
    bias2d = jnp.take(exp_bias1, genes_oi, axis=0).reshape(1, G)
    return _combine(pooled.reshape(CELLS, G), bias2d)
